# baseline (speedup 1.0000x reference)
import functools

import jax
import jax.numpy as jnp
from jax.experimental import pallas as pl
from jax.experimental.pallas import tpu as pltpu

HID = 128


def _mm_kernel(x_ref, w_ref, b_ref, o_ref, *, relu):
    acc = jnp.dot(x_ref[...], w_ref[...], preferred_element_type=jnp.float32)
    acc = acc + b_ref[...]
    if relu:
        acc = jnp.maximum(acc, 0.0)
    o_ref[...] = acc


def _mm(x, w, b, *, relu=False, block=1000):
    n = x.shape[0]
    assert n % block == 0, (n, block)
    grid = (n // block,)
    return pl.pallas_call(
        functools.partial(_mm_kernel, relu=relu),
        grid=grid,
        in_specs=[
            pl.BlockSpec((block, HID), lambda i: (i, 0)),
            pl.BlockSpec((HID, HID), lambda i: (0, 0)),
            pl.BlockSpec((1, HID), lambda i: (0, 0)),
        ],
        out_specs=pl.BlockSpec((block, HID), lambda i: (i, 0)),
        out_shape=jax.ShapeDtypeStruct((n, HID), jnp.float32),
    )(x, w, b.reshape(1, HID))


def _mm_res_kernel(a_ref, x_ref, w_ref, b_ref, o_ref):
    acc = jnp.dot(a_ref[...], w_ref[...], preferred_element_type=jnp.float32)
    acc = jnp.maximum(acc + b_ref[...], 0.0)
    o_ref[...] = x_ref[...] + acc


def _mm_residual_relu(agg, xv, w, b, *, block=1000):
    n = agg.shape[0]
    assert n % block == 0
    return pl.pallas_call(
        _mm_res_kernel,
        grid=(n // block,),
        in_specs=[
            pl.BlockSpec((block, HID), lambda i: (i, 0)),
            pl.BlockSpec((block, HID), lambda i: (i, 0)),
            pl.BlockSpec((HID, HID), lambda i: (0, 0)),
            pl.BlockSpec((1, HID), lambda i: (0, 0)),
        ],
        out_specs=pl.BlockSpec((block, HID), lambda i: (i, 0)),
        out_shape=jax.ShapeDtypeStruct((n, HID), jnp.float32),
    )(agg, xv, w, b.reshape(1, HID))


def _final_kernel(xn_ref, batch_ref, wp1_ref, bp1_ref, wp2_ref, bp2_ref, o_ref, acc_ref):
    i = pl.program_id(0)

    @pl.when(i == 0)
    def _():
        acc_ref[...] = jnp.zeros_like(acc_ref)

    b = batch_ref[0, 0]
    onehot = (b[:, None] == jax.lax.broadcasted_iota(jnp.int32, (1, 128), 1)).astype(jnp.float32)
    acc_ref[...] += jnp.dot(onehot.T, xn_ref[...], preferred_element_type=jnp.float32)

    @pl.when(i == pl.num_programs(0) - 1)
    def _():
        hg = acc_ref[...]
        h = jnp.maximum(jnp.dot(hg, wp1_ref[...], preferred_element_type=jnp.float32) + bp1_ref[...], 0.0)
        o_ref[...] = jnp.dot(h, wp2_ref[...], preferred_element_type=jnp.float32) + bp2_ref[...]


def _graph_head(xn, batch_clip, w_p1, b_p1, w_p2, b_p2, *, block=2000):
    n = xn.shape[0]
    assert n % block == 0
    return pl.pallas_call(
        _final_kernel,
        grid=(n // block,),
        in_specs=[
            pl.BlockSpec((block, HID), lambda i: (i, 0)),
            pl.BlockSpec((1, 1, block), lambda i: (i, 0, 0)),
            pl.BlockSpec((HID, HID), lambda i: (0, 0)),
            pl.BlockSpec((1, HID), lambda i: (0, 0)),
            pl.BlockSpec((HID, 1), lambda i: (0, 0)),
            pl.BlockSpec((1, 1), lambda i: (0, 0)),
        ],
        out_specs=pl.BlockSpec((128, 1), lambda i: (0, 0)),
        out_shape=jax.ShapeDtypeStruct((128, 1), jnp.float32),
        scratch_shapes=[pltpu.VMEM((128, HID), jnp.float32)],
    )(xn, batch_clip.reshape(n // block, 1, block), w_p1, b_p1.reshape(1, HID), w_p2, b_p2.reshape(1, 1))


def kernel(x, edge_attr, tuple_index, tuple_feat, msg_src, msg_dst, msg_edge, batch, num_graphs,
           x_emb, ea_emb, tf_emb, W_t0, b_t0, W_t1, b_t1, conv_W, conv_b,
           W_pool, b_pool, W_p1, b_p1, W_p2, b_p2):
    NT = tuple_index.shape[1]
    NV = x.shape[0]

    xe = x_emb[x]
    t0 = _mm(xe, W_t0, b_t0, block=400)
    t1 = _mm(xe, W_t1, b_t1, block=400)
    tf = tf_emb[tuple_feat]
    Xv = t0[tuple_index[0]] * t1[tuple_index[1]] * tf

    code = edge_attr[msg_edge]
    etab = ea_emb

    for l in range(6):
        msg = Xv[msg_src] * etab[code]
        agg = jax.ops.segment_sum(msg, msg_dst, num_segments=NT)
        Xv = _mm_residual_relu(agg, Xv, conv_W[l], conv_b[l], block=1000)

    sums = jax.ops.segment_sum(Xv, tuple_index[0], num_segments=NV)
    cnt = jax.ops.segment_sum(jnp.ones((NT,), jnp.float32), tuple_index[0], num_segments=NV)
    xn = sums / jnp.clip(cnt, 1.0)[:, None]
    xn = _mm(xn, W_pool, b_pool, relu=True, block=400)

    batch_clip = jnp.minimum(batch, num_graphs - 1).astype(jnp.int32)
    return _graph_head(xn, batch_clip, W_p1, b_p1, W_p2, b_p2, block=2000)


# speedup vs baseline: 1.2367x; 1.2367x over previous
import functools

import jax
import jax.numpy as jnp
from jax import lax
from jax.experimental import pallas as pl
from jax.experimental.pallas import tpu as pltpu
from jax.experimental.pallas import tpu_sc as plsc

HID = 128
NV = 10000
NT = 200000
NM = 600000

MB = 128
CHUNK = 632
NCHUNK = 320
NT_OUT = CHUNK * NCHUNK
CPT = 10
NM_PAD = 600064
NT_PAD = 200064
NTB = NT_PAD // MB
TB_PER_TILE = 49
CHUNK_L = 320
NV_PAD = CHUNK_L * 32

_mesh = plsc.VectorSubcoreMesh(core_axis_name="c", subcore_axis_name="s")


def _wid():
    return lax.axis_index("s") * 2 + lax.axis_index("c")


def _tupleinit_body(a0_hbm, a1_hbm, tf_hbm, codes_hbm, xv_hbm,
                    a0_v, a1_v, tf_v, codes_v, out_v):
    w = _wid()
    pltpu.sync_copy(a0_hbm, a0_v)
    pltpu.sync_copy(a1_hbm, a1_v)
    pltpu.sync_copy(tf_hbm, tf_v)
    for k in range(TB_PER_TILE):
        b = pl.multiple_of(w * TB_PER_TILE + k, 1)

        @pl.when(b < NTB)
        def _():
            pltpu.sync_copy(codes_hbm.at[pl.ds(pl.multiple_of(b * MB, MB), MB)], codes_v.at[pl.ds(0, MB)])

            def tbody(i, _):
                c = codes_v[pl.ds(i, 16)][0]
                c0 = c & 0xFF
                c1 = (c >> 8) & 0xFF
                c2 = c >> 16
                for j in range(8):
                    s = pl.ds(j * 16, 16)
                    out_v[i, s] = a0_v[c0, s] * a1_v[c1, s] * tf_v[c2, s]
                return _

            lax.fori_loop(0, MB, tbody, None)

            @pl.when(b < NTB - 1)
            def _():
                pltpu.sync_copy(out_v, xv_hbm.at[pl.ds(pl.multiple_of(b * MB, MB), MB)])

            @pl.when(b == NTB - 1)
            def _():
                pltpu.sync_copy(out_v.at[pl.ds(0, NT - (NTB - 1) * MB)],
                                xv_hbm.at[pl.ds(pl.multiple_of(b * MB, MB), NT - (NTB - 1) * MB)])


_tupleinit_sc = pl.kernel(
    _tupleinit_body,
    out_type=jax.ShapeDtypeStruct((NT, HID), jnp.float32),
    mesh=_mesh,
    scratch_types=[
        pltpu.VMEM((32, HID), jnp.float32),
        pltpu.VMEM((32, HID), jnp.float32),
        pltpu.VMEM((16, HID), jnp.float32),
        pltpu.VMEM((MB + 16,), jnp.int32),
        pltpu.VMEM((MB, HID), jnp.float32),
    ],
)


def _conv_body(xv_hbm, src_hbm, pk_hbm, off_hbm, etab_hbm, agg_hbm,
               etab_v, off_v, src_v, pk_v, rows_v, agg_v, sem):
    w = _wid()
    pltpu.sync_copy(etab_hbm, etab_v)
    pltpu.sync_copy(off_hbm, off_v.at[pl.ds(0, NCHUNK + 8)])
    for k in range(CPT):
        c = w * CPT + k

        def zbody(i, _):
            for j in range(8):
                agg_v[i, pl.ds(j * 16, 16)] = jnp.zeros((16,), jnp.float32)
            return _

        lax.fori_loop(0, CHUNK, zbody, None)

        mv = off_v[pl.ds(c, 16)]
        m0 = mv[0]
        m1 = mv[1]
        b0 = m0 >> 7
        nb = (m1 - (b0 << 7) + (MB - 1)) >> 7

        def bbody(b, _):
            g = pl.multiple_of((b0 + b) << 7, MB)
            pltpu.sync_copy(src_hbm.at[pl.ds(g, MB)], src_v.at[pl.ds(0, MB)])
            pltpu.sync_copy(pk_hbm.at[pl.ds(g, MB)], pk_v.at[pl.ds(0, MB)])
            pltpu.async_copy(xv_hbm.at[src_v.at[pl.ds(0, MB)]], rows_v, sem).wait()
            lo = jnp.maximum(m0 - g, 0)
            hi = jnp.minimum(m1 - g, MB)

            def mbody(i, _):
                v = pk_v[pl.ds(i, 16)][0]
                d = v & 0xFFFF
                cd = v >> 16
                for j in range(8):
                    s = pl.ds(j * 16, 16)
                    plsc.addupdate(agg_v.at[d, s], rows_v[i, s] * etab_v[cd, s])
                return _

            lax.fori_loop(lo, hi, mbody, None)
            return _

        lax.fori_loop(0, nb, bbody, None)
        pltpu.sync_copy(agg_v, agg_hbm.at[pl.ds(pl.multiple_of(c * CHUNK, 1), CHUNK)])


_conv_sc = pl.kernel(
    _conv_body,
    out_type=jax.ShapeDtypeStruct((NT_OUT, HID), jnp.float32),
    mesh=_mesh,
    scratch_types=[
        pltpu.VMEM((16, HID), jnp.float32),
        pltpu.VMEM((NCHUNK + 24,), jnp.int32),
        pltpu.VMEM((MB + 16,), jnp.int32),
        pltpu.VMEM((MB + 16,), jnp.int32),
        pltpu.VMEM((MB, HID), jnp.float32),
        pltpu.VMEM((CHUNK, HID), jnp.float32),
        pltpu.SemaphoreType.DMA,
    ],
)


def _lpool_body(xv_hbm, src_hbm, dstm_hbm, off_hbm, sums_hbm,
                off_v, src_v, dstm_v, rows_v, agg_v, sem):
    w = _wid()
    pltpu.sync_copy(off_hbm, off_v.at[pl.ds(0, 40)])

    def zbody(i, _):
        for j in range(8):
            agg_v[i, pl.ds(j * 16, 16)] = jnp.zeros((16,), jnp.float32)
        return _

    lax.fori_loop(0, CHUNK_L, zbody, None)

    mv = off_v[pl.ds(w, 16)]
    m0 = mv[0]
    m1 = mv[1]
    b0 = m0 >> 7
    nb = (m1 - (b0 << 7) + (MB - 1)) >> 7

    def bbody(b, _):
        g = pl.multiple_of((b0 + b) << 7, MB)
        pltpu.sync_copy(src_hbm.at[pl.ds(g, MB)], src_v.at[pl.ds(0, MB)])
        pltpu.sync_copy(dstm_hbm.at[pl.ds(g, MB)], dstm_v.at[pl.ds(0, MB)])
        pltpu.async_copy(xv_hbm.at[src_v.at[pl.ds(0, MB)]], rows_v, sem).wait()
        lo = jnp.maximum(m0 - g, 0)
        hi = jnp.minimum(m1 - g, MB)

        def mbody(i, _):
            d = dstm_v[pl.ds(i, 16)][0]
            for j in range(8):
                s = pl.ds(j * 16, 16)
                plsc.addupdate(agg_v.at[d, s], rows_v[i, s])
            return _

        lax.fori_loop(lo, hi, mbody, None)
        return _

    lax.fori_loop(0, nb, bbody, None)
    pltpu.sync_copy(agg_v, sums_hbm.at[pl.ds(w * CHUNK_L, CHUNK_L)])


_lpool_sc = pl.kernel(
    _lpool_body,
    out_type=jax.ShapeDtypeStruct((NV_PAD, HID), jnp.float32),
    mesh=_mesh,
    scratch_types=[
        pltpu.VMEM((56,), jnp.int32),
        pltpu.VMEM((MB + 16,), jnp.int32),
        pltpu.VMEM((MB + 16,), jnp.int32),
        pltpu.VMEM((MB, HID), jnp.float32),
        pltpu.VMEM((CHUNK_L, HID), jnp.float32),
        pltpu.SemaphoreType.DMA,
    ],
)


def _pre_tables_kernel(xe_ref, w0_ref, b0_ref, w1_ref, b1_ref, a0_ref, a1_ref):
    a0_ref[...] = jnp.dot(xe_ref[...], w0_ref[...], preferred_element_type=jnp.float32) + b0_ref[...]
    a1_ref[...] = jnp.dot(xe_ref[...], w1_ref[...], preferred_element_type=jnp.float32) + b1_ref[...]


def _pre_tables(x_emb, W_t0, b_t0, W_t1, b_t1):
    full = pl.BlockSpec((32, HID), lambda: (0, 0))
    wspec = pl.BlockSpec((HID, HID), lambda: (0, 0))
    bspec = pl.BlockSpec((1, HID), lambda: (0, 0))
    return pl.pallas_call(
        _pre_tables_kernel,
        in_specs=[full, wspec, bspec, wspec, bspec],
        out_specs=[full, full],
        out_shape=[jax.ShapeDtypeStruct((32, HID), jnp.float32)] * 2,
    )(x_emb, W_t0, b_t0.reshape(1, HID), W_t1, b_t1.reshape(1, HID))


def _mm_res_kernel(a_ref, x_ref, w_ref, b_ref, o_ref):
    acc = jnp.dot(a_ref[...], w_ref[...], preferred_element_type=jnp.float32)
    acc = jnp.maximum(acc + b_ref[...], 0.0)
    o_ref[...] = x_ref[...] + acc


def _mm_residual_relu(agg, xv, w, b, *, block=1000):
    n = xv.shape[0]
    assert n % block == 0
    return pl.pallas_call(
        _mm_res_kernel,
        grid=(n // block,),
        in_specs=[
            pl.BlockSpec((block, HID), lambda i: (i, 0)),
            pl.BlockSpec((block, HID), lambda i: (i, 0)),
            pl.BlockSpec((HID, HID), lambda i: (0, 0)),
            pl.BlockSpec((1, HID), lambda i: (0, 0)),
        ],
        out_specs=pl.BlockSpec((block, HID), lambda i: (i, 0)),
        out_shape=jax.ShapeDtypeStruct((n, HID), jnp.float32),
    )(agg, xv, w, b.reshape(1, HID))


def _pool_kernel(s_ref, c_ref, w_ref, b_ref, o_ref):
    xn = s_ref[...] / c_ref[...]
    acc = jnp.dot(xn, w_ref[...], preferred_element_type=jnp.float32)
    o_ref[...] = jnp.maximum(acc + b_ref[...], 0.0)


def _pool_mm(sums, cnt, w, b, *, block=400):
    return pl.pallas_call(
        _pool_kernel,
        grid=(NV // block,),
        in_specs=[
            pl.BlockSpec((block, HID), lambda i: (i, 0)),
            pl.BlockSpec((block, 1), lambda i: (i, 0)),
            pl.BlockSpec((HID, HID), lambda i: (0, 0)),
            pl.BlockSpec((1, HID), lambda i: (0, 0)),
        ],
        out_specs=pl.BlockSpec((block, HID), lambda i: (i, 0)),
        out_shape=jax.ShapeDtypeStruct((NV, HID), jnp.float32),
    )(sums, cnt.reshape(-1, 1), w, b.reshape(1, HID))


def _final_kernel(xn_ref, batch_ref, wp1_ref, bp1_ref, wp2_ref, bp2_ref, o_ref, acc_ref):
    i = pl.program_id(0)

    @pl.when(i == 0)
    def _():
        acc_ref[...] = jnp.zeros_like(acc_ref)

    b = batch_ref[0, 0]
    onehot = (b[:, None] == jax.lax.broadcasted_iota(jnp.int32, (1, 128), 1)).astype(jnp.float32)
    acc_ref[...] += jax.lax.dot(onehot.T, xn_ref[...], precision=jax.lax.Precision.HIGHEST,
                                preferred_element_type=jnp.float32)

    @pl.when(i == pl.num_programs(0) - 1)
    def _():
        hg = acc_ref[...]
        h = jnp.maximum(jnp.dot(hg, wp1_ref[...], preferred_element_type=jnp.float32) + bp1_ref[...], 0.0)
        o_ref[...] = jnp.dot(h, wp2_ref[...], preferred_element_type=jnp.float32) + bp2_ref[...]


def _graph_head(xn, batch_clip, w_p1, b_p1, w_p2, b_p2, *, block=2000):
    n = xn.shape[0]
    return pl.pallas_call(
        _final_kernel,
        grid=(n // block,),
        in_specs=[
            pl.BlockSpec((block, HID), lambda i: (i, 0)),
            pl.BlockSpec((1, 1, block), lambda i: (i, 0, 0)),
            pl.BlockSpec((HID, HID), lambda i: (0, 0)),
            pl.BlockSpec((1, HID), lambda i: (0, 0)),
            pl.BlockSpec((HID, 1), lambda i: (0, 0)),
            pl.BlockSpec((1, 1), lambda i: (0, 0)),
        ],
        out_specs=pl.BlockSpec((128, 1), lambda i: (0, 0)),
        out_shape=jax.ShapeDtypeStruct((128, 1), jnp.float32),
        scratch_shapes=[pltpu.VMEM((128, HID), jnp.float32)],
    )(xn, batch_clip.reshape(n // block, 1, block), w_p1, b_p1.reshape(1, HID), w_p2, b_p2.reshape(1, 1))


def _pad1d(a, n, fill=0):
    return jnp.concatenate([a, jnp.full((n - a.shape[0],), fill, a.dtype)])


def kernel(x, edge_attr, tuple_index, tuple_feat, msg_src, msg_dst, msg_edge, batch, num_graphs,
           x_emb, ea_emb, tf_emb, W_t0, b_t0, W_t1, b_t1, conv_W, conv_b,
           W_pool, b_pool, W_p1, b_p1, W_p2, b_p2):
    i32 = jnp.int32

    codes_t = (x[tuple_index[0]].astype(i32)
               | (x[tuple_index[1]].astype(i32) << 8)
               | (tuple_feat.astype(i32) << 16))
    codes_t = _pad1d(codes_t, NT_PAD)

    scode = edge_attr[msg_edge].astype(i32)
    perm = jnp.argsort(msg_dst)
    sdst = msg_dst[perm].astype(i32)
    ssrc = _pad1d(msg_src[perm].astype(i32), NM_PAD)
    spk = _pad1d((sdst % CHUNK) | (scode[perm] << 16), NM_PAD)
    off = _pad1d(jnp.searchsorted(sdst, jnp.arange(0, NT + 1, CHUNK, dtype=i32)).astype(i32),
                 NCHUNK + 8, NM)

    ti0 = tuple_index[0].astype(i32)
    tperm = jnp.argsort(ti0)
    sti0 = ti0[tperm]
    tsrc = _pad1d(tperm.astype(i32), NT_PAD)
    tdstm = _pad1d(sti0 % CHUNK_L, NT_PAD)
    bounds = jnp.minimum(jnp.arange(33, dtype=i32) * CHUNK_L, NV)
    off_l = _pad1d(jnp.searchsorted(sti0, bounds).astype(i32), 40, NT)
    nodeoff = jnp.searchsorted(sti0, jnp.arange(NV + 1, dtype=i32))
    cnt = jnp.maximum(jnp.diff(nodeoff).astype(jnp.float32), 1.0)

    batch_clip = jnp.minimum(batch, num_graphs - 1).astype(i32)

    A0, A1 = _pre_tables(x_emb, W_t0, b_t0, W_t1, b_t1)
    Xv = _tupleinit_sc(A0, A1, tf_emb, codes_t)

    for l in range(6):
        agg = _conv_sc(Xv, ssrc, spk, off, ea_emb)
        Xv = _mm_residual_relu(agg, Xv, conv_W[l], conv_b[l], block=1000)

    sums = _lpool_sc(Xv, tsrc, tdstm, off_l)
    xn = _pool_mm(sums, cnt, W_pool, b_pool, block=400)
    return _graph_head(xn, batch_clip, W_p1, b_p1, W_p2, b_p2, block=2000)


# speedup vs baseline: 1.4318x; 1.1577x over previous
import functools

import jax
import jax.numpy as jnp
from jax import lax
from jax.experimental import pallas as pl
from jax.experimental.pallas import tpu as pltpu
from jax.experimental.pallas import tpu_sc as plsc

HID = 128
NV = 10000
NT = 200000
NM = 600000

MB = 256
NB = 2344
NM_PAD = NB * MB
CHUNK = 448
NCHUNK = 447
NT_OUT = NCHUNK * CHUNK
CPT = 14
PKS = MB + 16

TIB = 1024
NT_PAD_T = 200704

CHUNK_L = 320
NV_PAD = CHUNK_L * 32
NBL = 782
NTL_PAD = NBL * MB

_mesh = plsc.VectorSubcoreMesh(core_axis_name="c", subcore_axis_name="s")


def _wid():
    return lax.axis_index("s") * 2 + lax.axis_index("c")


def _conv_body(xv_hbm, idx_hbm, off_hbm, etab_hbm, agg_hbm,
               etab_v, off_v, src_v, pk_v, rows_v, agg_v, semi, semp, semg):
    w = _wid()
    pltpu.sync_copy(etab_hbm, etab_v)
    pltpu.sync_copy(off_hbm, off_v)

    def idx_start(gb, slot):
        base = pl.multiple_of(gb * (2 * MB), 8)
        pltpu.async_copy(idx_hbm.at[pl.ds(base, MB)], src_v.at[slot], semi.at[slot])
        pltpu.async_copy(idx_hbm.at[pl.ds(base + MB, MB)],
                         pk_v.at[pl.ds(pl.multiple_of(slot * PKS, 8), MB)], semp.at[slot])

    def idx_wait(gb, slot):
        base = pl.multiple_of(gb * (2 * MB), 8)
        pltpu.make_async_copy(idx_hbm.at[pl.ds(base, MB)], src_v.at[slot], semi.at[slot]).wait()
        pltpu.make_async_copy(idx_hbm.at[pl.ds(base + MB, MB)],
                              pk_v.at[pl.ds(pl.multiple_of(slot * PKS, 8), MB)], semp.at[slot]).wait()

    def gather_start(gb, islot, r):
        for h in range(2):
            pltpu.async_copy(xv_hbm.at[src_v.at[islot, pl.ds(h * 128, 128)]],
                             rows_v.at[r, pl.ds(h * 128, 128)], semg.at[r])

    def gather_wait(gb, islot, r):
        for h in range(2):
            pltpu.make_async_copy(xv_hbm.at[src_v.at[islot, pl.ds(h * 128, 128)]],
                                  rows_v.at[r, pl.ds(h * 128, 128)], semg.at[r]).wait()

    for k in range(CPT):
        c = w + 32 * k

        @pl.when(c < NCHUNK)
        def _():
            def zbody(i, _):
                for j in range(8):
                    agg_v[i, pl.ds(j * 16, 16)] = jnp.zeros((16,), jnp.float32)
                return _

            lax.fori_loop(0, CHUNK, zbody, None)

            mv = off_v[pl.ds(c, 16)]
            m0 = mv[0]
            m1 = mv[1]
            b0 = m0 >> 8
            nb = (m1 - (b0 << 8) + (MB - 1)) >> 8

            @pl.when(nb > 0)
            def _():
                idx_start(b0, 0)
                for p in (1, 2):
                    @pl.when(nb > p)
                    def _():
                        idx_start(b0 + p, p)
                idx_wait(b0, 0)
                gather_start(b0, 0, 0)

                def bbody(b, _):
                    r = b & 1
                    islot = b & 3

                    @pl.when(b + 1 < nb)
                    def _():
                        idx_wait(b0 + b + 1, (b + 1) & 3)
                        gather_start(b0 + b + 1, (b + 1) & 3, r ^ 1)

                    @pl.when(b + 3 < nb)
                    def _():
                        idx_start(b0 + b + 3, (b + 3) & 3)

                    gather_wait(b0 + b, islot, r)
                    g = (b0 + b) << 8
                    lo = jnp.maximum(m0 - g, 0)
                    hi = jnp.minimum(m1 - g, MB)

                    pkb = pl.multiple_of(islot * PKS, 8)

                    def mbody(i, _):
                        v = pk_v[pl.ds(pkb + i, 16)][0]
                        d = v & 0xFFFF
                        cd = v >> 16
                        for j in range(8):
                            s = pl.ds(j * 16, 16)
                            plsc.addupdate(agg_v.at[d, s], rows_v[r, i, s] * etab_v[cd, s])
                        return _

                    lax.fori_loop(lo, hi, mbody, None)
                    return _

                lax.fori_loop(0, nb, bbody, None)

            pltpu.sync_copy(agg_v, agg_hbm.at[pl.ds(pl.multiple_of(c * CHUNK, 8), CHUNK)])


_conv_sc = pl.kernel(
    _conv_body,
    out_type=jax.ShapeDtypeStruct((NT_OUT, HID), jnp.float32),
    mesh=_mesh,
    scratch_types=[
        pltpu.VMEM((16, HID), jnp.float32),
        pltpu.VMEM((NCHUNK + 25,), jnp.int32),
        pltpu.VMEM((4, MB), jnp.int32),
        pltpu.VMEM((4 * PKS,), jnp.int32),
        pltpu.VMEM((2, MB, HID), jnp.float32),
        pltpu.VMEM((CHUNK, HID), jnp.float32),
        pltpu.SemaphoreType.DMA((4,)),
        pltpu.SemaphoreType.DMA((4,)),
        pltpu.SemaphoreType.DMA((2,)),
    ],
)


def _lpool_body(xv_hbm, idx_hbm, off_hbm, sums_hbm,
                off_v, src_v, pk_v, rows_v, agg_v, semi, semp, semg):
    w = _wid()
    pltpu.sync_copy(off_hbm, off_v)

    def idx_start(gb, slot):
        base = pl.multiple_of(gb * (2 * MB), 8)
        pltpu.async_copy(idx_hbm.at[pl.ds(base, MB)], src_v.at[slot], semi.at[slot])
        pltpu.async_copy(idx_hbm.at[pl.ds(base + MB, MB)],
                         pk_v.at[pl.ds(pl.multiple_of(slot * PKS, 8), MB)], semp.at[slot])

    def idx_wait(gb, slot):
        base = pl.multiple_of(gb * (2 * MB), 8)
        pltpu.make_async_copy(idx_hbm.at[pl.ds(base, MB)], src_v.at[slot], semi.at[slot]).wait()
        pltpu.make_async_copy(idx_hbm.at[pl.ds(base + MB, MB)],
                              pk_v.at[pl.ds(pl.multiple_of(slot * PKS, 8), MB)], semp.at[slot]).wait()

    def gather_start(gb, islot, r):
        for h in range(2):
            pltpu.async_copy(xv_hbm.at[src_v.at[islot, pl.ds(h * 128, 128)]],
                             rows_v.at[r, pl.ds(h * 128, 128)], semg.at[r])

    def gather_wait(gb, islot, r):
        for h in range(2):
            pltpu.make_async_copy(xv_hbm.at[src_v.at[islot, pl.ds(h * 128, 128)]],
                                  rows_v.at[r, pl.ds(h * 128, 128)], semg.at[r]).wait()

    def zbody(i, _):
        for j in range(8):
            agg_v[i, pl.ds(j * 16, 16)] = jnp.zeros((16,), jnp.float32)
        return _

    lax.fori_loop(0, CHUNK_L, zbody, None)

    mv = off_v[pl.ds(w, 16)]
    m0 = mv[0]
    m1 = mv[1]
    b0 = m0 >> 8
    nb = (m1 - (b0 << 8) + (MB - 1)) >> 8

    @pl.when(nb > 0)
    def _():
        idx_start(b0, 0)
        for p in (1, 2):
            @pl.when(nb > p)
            def _():
                idx_start(b0 + p, p)
        idx_wait(b0, 0)
        gather_start(b0, 0, 0)

        def bbody(b, _):
            r = b & 1
            islot = b & 3

            @pl.when(b + 1 < nb)
            def _():
                idx_wait(b0 + b + 1, (b + 1) & 3)
                gather_start(b0 + b + 1, (b + 1) & 3, r ^ 1)

            @pl.when(b + 3 < nb)
            def _():
                idx_start(b0 + b + 3, (b + 3) & 3)

            gather_wait(b0 + b, islot, r)
            g = (b0 + b) << 8
            lo = jnp.maximum(m0 - g, 0)
            hi = jnp.minimum(m1 - g, MB)

            pkb = pl.multiple_of(islot * PKS, 8)

            def mbody(i, _):
                d = pk_v[pl.ds(pkb + i, 16)][0]
                for j in range(8):
                    s = pl.ds(j * 16, 16)
                    plsc.addupdate(agg_v.at[d, s], rows_v[r, i, s])
                return _

            lax.fori_loop(lo, hi, mbody, None)
            return _

        lax.fori_loop(0, nb, bbody, None)

    pltpu.sync_copy(agg_v, sums_hbm.at[pl.ds(pl.multiple_of(w * CHUNK_L, 8), CHUNK_L)])


_lpool_sc = pl.kernel(
    _lpool_body,
    out_type=jax.ShapeDtypeStruct((NV_PAD, HID), jnp.float32),
    mesh=_mesh,
    scratch_types=[
        pltpu.VMEM((56,), jnp.int32),
        pltpu.VMEM((4, MB), jnp.int32),
        pltpu.VMEM((4 * PKS,), jnp.int32),
        pltpu.VMEM((2, MB, HID), jnp.float32),
        pltpu.VMEM((CHUNK_L, HID), jnp.float32),
        pltpu.SemaphoreType.DMA((4,)),
        pltpu.SemaphoreType.DMA((4,)),
        pltpu.SemaphoreType.DMA((2,)),
    ],
)


_HI = jax.lax.Precision.HIGHEST


def _tupleinit_kernel(codes_ref, a0_ref, a1_ref, tf_ref, o_ref):
    c = codes_ref[0, 0]
    c0 = c & 0xFF
    c1 = (c >> 8) & 0xFF
    c2 = c >> 16
    i32 = jnp.int32
    oh0 = (c0[:, None] == jax.lax.broadcasted_iota(i32, (1, 32), 1)).astype(jnp.float32)
    oh1 = (c1[:, None] == jax.lax.broadcasted_iota(i32, (1, 32), 1)).astype(jnp.float32)
    oh2 = (c2[:, None] == jax.lax.broadcasted_iota(i32, (1, 16), 1)).astype(jnp.float32)
    v0 = jax.lax.dot(oh0, a0_ref[...], precision=_HI, preferred_element_type=jnp.float32)
    v1 = jax.lax.dot(oh1, a1_ref[...], precision=_HI, preferred_element_type=jnp.float32)
    v2 = jax.lax.dot(oh2, tf_ref[...], precision=_HI, preferred_element_type=jnp.float32)
    o_ref[...] = v0 * v1 * v2


def _tupleinit_tc(codes, A0, A1, tf_emb):
    nblk = NT_PAD_T // TIB
    return pl.pallas_call(
        _tupleinit_kernel,
        grid=(nblk,),
        in_specs=[
            pl.BlockSpec((1, 1, TIB), lambda i: (i, 0, 0)),
            pl.BlockSpec((32, HID), lambda i: (0, 0)),
            pl.BlockSpec((32, HID), lambda i: (0, 0)),
            pl.BlockSpec((16, HID), lambda i: (0, 0)),
        ],
        out_specs=pl.BlockSpec((TIB, HID), lambda i: (i, 0)),
        out_shape=jax.ShapeDtypeStruct((NT_PAD_T, HID), jnp.float32),
    )(codes.reshape(nblk, 1, TIB), A0, A1, tf_emb)


def _pre_tables_kernel(xe_ref, w0_ref, b0_ref, w1_ref, b1_ref, a0_ref, a1_ref):
    a0_ref[...] = jnp.dot(xe_ref[...], w0_ref[...], preferred_element_type=jnp.float32) + b0_ref[...]
    a1_ref[...] = jnp.dot(xe_ref[...], w1_ref[...], preferred_element_type=jnp.float32) + b1_ref[...]


def _pre_tables(x_emb, W_t0, b_t0, W_t1, b_t1):
    full = pl.BlockSpec((32, HID), lambda: (0, 0))
    wspec = pl.BlockSpec((HID, HID), lambda: (0, 0))
    bspec = pl.BlockSpec((1, HID), lambda: (0, 0))
    return pl.pallas_call(
        _pre_tables_kernel,
        in_specs=[full, wspec, bspec, wspec, bspec],
        out_specs=[full, full],
        out_shape=[jax.ShapeDtypeStruct((32, HID), jnp.float32)] * 2,
    )(x_emb, W_t0, b_t0.reshape(1, HID), W_t1, b_t1.reshape(1, HID))


def _mm_res_kernel(a_ref, x_ref, w_ref, b_ref, o_ref):
    acc = jnp.dot(a_ref[...], w_ref[...], preferred_element_type=jnp.float32)
    acc = jnp.maximum(acc + b_ref[...], 0.0)
    o_ref[...] = x_ref[...] + acc


def _mm_residual_relu(agg, xv, w, b, *, block=1000):
    return pl.pallas_call(
        _mm_res_kernel,
        grid=(NT // block,),
        in_specs=[
            pl.BlockSpec((block, HID), lambda i: (i, 0)),
            pl.BlockSpec((block, HID), lambda i: (i, 0)),
            pl.BlockSpec((HID, HID), lambda i: (0, 0)),
            pl.BlockSpec((1, HID), lambda i: (0, 0)),
        ],
        out_specs=pl.BlockSpec((block, HID), lambda i: (i, 0)),
        out_shape=jax.ShapeDtypeStruct((NT, HID), jnp.float32),
    )(agg, xv, w, b.reshape(1, HID))


def _pool_kernel(s_ref, c_ref, w_ref, b_ref, o_ref):
    xn = s_ref[...] / c_ref[...]
    acc = jnp.dot(xn, w_ref[...], preferred_element_type=jnp.float32)
    o_ref[...] = jnp.maximum(acc + b_ref[...], 0.0)


def _pool_mm(sums, cnt, w, b, *, block=400):
    return pl.pallas_call(
        _pool_kernel,
        grid=(NV // block,),
        in_specs=[
            pl.BlockSpec((block, HID), lambda i: (i, 0)),
            pl.BlockSpec((block, 1), lambda i: (i, 0)),
            pl.BlockSpec((HID, HID), lambda i: (0, 0)),
            pl.BlockSpec((1, HID), lambda i: (0, 0)),
        ],
        out_specs=pl.BlockSpec((block, HID), lambda i: (i, 0)),
        out_shape=jax.ShapeDtypeStruct((NV, HID), jnp.float32),
    )(sums, cnt.reshape(-1, 1), w, b.reshape(1, HID))


def _final_kernel(xn_ref, batch_ref, wp1_ref, bp1_ref, wp2_ref, bp2_ref, o_ref, acc_ref):
    i = pl.program_id(0)

    @pl.when(i == 0)
    def _():
        acc_ref[...] = jnp.zeros_like(acc_ref)

    b = batch_ref[0, 0]
    onehot = (b[:, None] == jax.lax.broadcasted_iota(jnp.int32, (1, 128), 1)).astype(jnp.float32)
    acc_ref[...] += jax.lax.dot(onehot.T, xn_ref[...], precision=_HI,
                                preferred_element_type=jnp.float32)

    @pl.when(i == pl.num_programs(0) - 1)
    def _():
        hg = acc_ref[...]
        h = jnp.maximum(jnp.dot(hg, wp1_ref[...], preferred_element_type=jnp.float32) + bp1_ref[...], 0.0)
        o_ref[...] = jnp.dot(h, wp2_ref[...], preferred_element_type=jnp.float32) + bp2_ref[...]


def _graph_head(xn, batch_clip, w_p1, b_p1, w_p2, b_p2, *, block=2000):
    n = xn.shape[0]
    return pl.pallas_call(
        _final_kernel,
        grid=(n // block,),
        in_specs=[
            pl.BlockSpec((block, HID), lambda i: (i, 0)),
            pl.BlockSpec((1, 1, block), lambda i: (i, 0, 0)),
            pl.BlockSpec((HID, HID), lambda i: (0, 0)),
            pl.BlockSpec((1, HID), lambda i: (0, 0)),
            pl.BlockSpec((HID, 1), lambda i: (0, 0)),
            pl.BlockSpec((1, 1), lambda i: (0, 0)),
        ],
        out_specs=pl.BlockSpec((128, 1), lambda i: (0, 0)),
        out_shape=jax.ShapeDtypeStruct((128, 1), jnp.float32),
        scratch_shapes=[pltpu.VMEM((128, HID), jnp.float32)],
    )(xn, batch_clip.reshape(n // block, 1, block), w_p1, b_p1.reshape(1, HID), w_p2, b_p2.reshape(1, 1))


def _pad1d(a, n, fill=0):
    return jnp.concatenate([a, jnp.full((n - a.shape[0],), fill, a.dtype)])


def kernel(x, edge_attr, tuple_index, tuple_feat, msg_src, msg_dst, msg_edge, batch, num_graphs,
           x_emb, ea_emb, tf_emb, W_t0, b_t0, W_t1, b_t1, conv_W, conv_b,
           W_pool, b_pool, W_p1, b_p1, W_p2, b_p2):
    i32 = jnp.int32
    u64 = jnp.uint64

    codes_t = (x[tuple_index[0]].astype(i32)
               | (x[tuple_index[1]].astype(i32) << 8)
               | (tuple_feat.astype(i32) << 16))
    codes_t = _pad1d(codes_t, NT_PAD_T)

    scode = edge_attr[msg_edge].astype(i32)
    val = msg_src.astype(i32) | (scode << 18)
    sdst, sval = jax.lax.sort((msg_dst.astype(i32), val), num_keys=1)
    ssrc = sval & 0x3FFFF
    spk = (sdst % CHUNK) | (((sval >> 18) & 0xF) << 16)
    midx = jnp.stack([_pad1d(ssrc, NM_PAD).reshape(NB, MB),
                      _pad1d(spk, NM_PAD).reshape(NB, MB)], axis=1).reshape(-1)
    off = _pad1d(jnp.searchsorted(sdst, jnp.arange(NCHUNK + 1, dtype=i32) * CHUNK).astype(i32),
                 NCHUNK + 25, NM)

    keyt = (tuple_index[0].astype(jnp.uint32) << jnp.uint32(18)) | jnp.arange(NT, dtype=jnp.uint32)
    keyt = jnp.sort(keyt)
    sti0 = (keyt >> jnp.uint32(18)).astype(i32)
    tsrc = (keyt & jnp.uint32(0x3FFFF)).astype(i32)
    tidx = jnp.stack([_pad1d(tsrc, NTL_PAD).reshape(NBL, MB),
                      _pad1d(sti0 % CHUNK_L, NTL_PAD).reshape(NBL, MB)], axis=1).reshape(-1)
    bounds = jnp.minimum(jnp.arange(33, dtype=i32) * CHUNK_L, NV)
    off_l = _pad1d(jnp.searchsorted(sti0, bounds).astype(i32), 56, NT)
    nodeoff = jnp.searchsorted(sti0, jnp.arange(NV + 1, dtype=i32))
    cnt = jnp.maximum(jnp.diff(nodeoff).astype(jnp.float32), 1.0)

    batch_clip = jnp.minimum(batch, num_graphs - 1).astype(i32)

    A0, A1 = _pre_tables(x_emb, W_t0, b_t0, W_t1, b_t1)
    Xv = _tupleinit_tc(codes_t, A0, A1, tf_emb)

    for l in range(6):
        agg = _conv_sc(Xv, midx, off, ea_emb)
        Xv = _mm_residual_relu(agg, Xv, conv_W[l], conv_b[l], block=1000)

    sums = _lpool_sc(Xv, tidx, off_l)
    xn = _pool_mm(sums, cnt, W_pool, b_pool, block=400)
    return _graph_head(xn, batch_clip, W_p1, b_p1, W_p2, b_p2, block=2000)
